# Initial kernel scaffold; baseline (speedup 1.0000x reference)
import jax
import jax.numpy as jnp
from jax.experimental import pallas as pl


def kernel(direct_cls_logits, inner_product, cls_labels):
    raise NotImplementedError("write your pallas kernel here")


# speedup vs baseline: 7.0003x; 7.0003x over previous
import functools

import jax
import jax.numpy as jnp
from jax import lax
from jax.experimental import pallas as pl
from jax.experimental.pallas import tpu as pltpu
from jax.experimental.pallas import tpu_sc as plsc

NUM_STEPS = 129
CLASS_NUM = 64
SUPPORT_NUM = 320
TARGET_NUM = 1024
INV_STEP = 64.0

NC, NS = 2, 16
NW = NC * NS
ROWS_W = TARGET_NUM // NW
VALS_W = ROWS_W * SUPPORT_NUM
VECS_ROW = SUPPORT_NUM // 16
HIST_W = 2 * NUM_STEPS * 16


def _sc_hist_body(ip_hbm, lab_hbm, out_hbm, sbuf, slab, tlab, hist):
    wid = lax.axis_index("s") * NC + lax.axis_index("c")
    pltpu.sync_copy(ip_hbm.at[pl.ds(wid * VALS_W, VALS_W)], sbuf)
    pltpu.sync_copy(lab_hbm.at[pl.ds(0, SUPPORT_NUM)], slab)
    pltpu.sync_copy(lab_hbm.at[pl.ds(SUPPORT_NUM + wid * ROWS_W, ROWS_W)],
                    tlab.at[pl.ds(0, ROWS_W)])

    zeros16 = jnp.zeros((16,), jnp.float32)

    def zbody(z, c):
        hist[pl.ds(z * 16, 16)] = zeros16
        return c

    lax.fori_loop(0, HIST_W // 16, zbody, 0)

    lane = lax.broadcasted_iota(jnp.int32, (16,), 0)

    def row_body(i, c):
        tl = tlab[pl.ds(i, 16)][0]

        def vec_body(v, c2_):
            s = sbuf[pl.ds(i * SUPPORT_NUM + v * 16, 16)]
            u = s * INV_STEP
            bi = u.astype(jnp.int32)
            bf = bi.astype(jnp.float32)
            frac = u - bf
            rem = (bf + 1.0) - u
            binv = jnp.minimum(jnp.maximum(bi + 64, 0), NUM_STEPS - 1)
            sl = slab[pl.ds(v * 16, 16)]
            pos = sl == tl
            base = jnp.where(pos, 0, NUM_STEPS * 16) + binv * 16 + lane
            plsc.addupdate_scatter(hist, [base], frac)
            plsc.addupdate_scatter(hist, [base + 16], rem,
                                   mask=binv < NUM_STEPS - 1)
            return c2_

        lax.fori_loop(0, VECS_ROW, vec_body, c)
        return c

    lax.fori_loop(0, ROWS_W, row_body, 0)
    pltpu.sync_copy(hist, out_hbm.at[wid])


@functools.cache
def _sc_hist():
    return pl.kernel(
        _sc_hist_body,
        out_type=jax.ShapeDtypeStruct((NW, HIST_W), jnp.float32),
        mesh=plsc.VectorSubcoreMesh(
            core_axis_name="c", subcore_axis_name="s",
            num_cores=NC, num_subcores=NS),
        scratch_types=[
            pltpu.VMEM((VALS_W,), jnp.float32),
            pltpu.VMEM((SUPPORT_NUM,), jnp.int32),
            pltpu.VMEM((ROWS_W + 16,), jnp.int32),
            pltpu.VMEM((HIST_W,), jnp.float32),
        ],
        compiler_params=pltpu.CompilerParams(needs_layout_passes=False),
    )


def _tc_combine_body(logits_ref, tl_ref, sl_ref, part_ref, hloss_ref, dloss_ref):
    x = logits_ref[...]
    m = jnp.max(x, axis=1, keepdims=True)
    e = jnp.exp(x - m)
    lse = jnp.log(jnp.sum(e, axis=1, keepdims=True)) + m
    cls = lax.broadcasted_iota(jnp.int32, (TARGET_NUM, CLASS_NUM), 1)
    sel = cls == tl_ref[...]
    picked = jnp.sum(jnp.where(sel, x - lse, 0.0))
    dloss_ref[...] = jnp.full((1, 1), -picked / TARGET_NUM, jnp.float32)

    tcnt = jnp.sum(sel.astype(jnp.float32), axis=0, keepdims=True)
    scls = lax.broadcasted_iota(jnp.int32, (CLASS_NUM, SUPPORT_NUM), 0)
    scnt = jnp.sum((scls == sl_ref[...]).astype(jnp.float32), axis=1,
                   keepdims=True)
    pos_num = jnp.dot(tcnt, scnt, preferred_element_type=jnp.float32)[0, 0]
    neg_num = float(TARGET_NUM * SUPPORT_NUM) - pos_num

    h = jnp.sum(jnp.sum(part_ref[...], axis=0), axis=1, keepdims=True)
    hpos = h[:NUM_STEPS] / pos_num
    hneg = h[NUM_STEPS:] / neg_num
    r = lax.broadcasted_iota(jnp.int32, (NUM_STEPS, NUM_STEPS), 0)
    c = lax.broadcasted_iota(jnp.int32, (NUM_STEPS, NUM_STEPS), 1)
    tri = (r >= c).astype(jnp.float32)
    cdf = jnp.dot(tri, hpos, preferred_element_type=jnp.float32)
    hloss_ref[...] = jnp.full((1, 1), jnp.sum(cdf * hneg), jnp.float32)


_tc_combine = pl.pallas_call(
    _tc_combine_body,
    out_shape=(
        jax.ShapeDtypeStruct((1, 1), jnp.float32),
        jax.ShapeDtypeStruct((1, 1), jnp.float32),
    ),
)


def kernel(direct_cls_logits, inner_product, cls_labels):
    labels = cls_labels.astype(jnp.int32)
    part = _sc_hist()(inner_product.reshape(-1), labels)
    part = part.reshape(NW, 2 * NUM_STEPS, 16)
    tl = labels[SUPPORT_NUM:].reshape(TARGET_NUM, 1)
    sl = labels[:SUPPORT_NUM].reshape(1, SUPPORT_NUM)
    hloss, dloss = _tc_combine(direct_cls_logits, tl, sl, part)
    return hloss[0, 0], dloss[0, 0]


# speedup vs baseline: 7.3816x; 1.0545x over previous
import functools

import jax
import jax.numpy as jnp
from jax import lax
from jax.experimental import pallas as pl
from jax.experimental.pallas import tpu as pltpu
from jax.experimental.pallas import tpu_sc as plsc

NUM_STEPS = 129
CLASS_NUM = 64
SUPPORT_NUM = 320
TARGET_NUM = 1024
INV_STEP = 64.0

NC, NS = 2, 16
NW = NC * NS
ROWS_W = TARGET_NUM // NW
VALS_W = ROWS_W * SUPPORT_NUM
VECS_ROW = SUPPORT_NUM // 16
HIST_W = 2 * NUM_STEPS * 16


def _sc_hist_body(ip_hbm, lab_hbm, out_hbm, sbuf, slab, tlab, hist):
    wid = lax.axis_index("s") * NC + lax.axis_index("c")
    pltpu.sync_copy(ip_hbm.at[pl.ds(wid * VALS_W, VALS_W)], sbuf)
    pltpu.sync_copy(lab_hbm.at[pl.ds(0, SUPPORT_NUM)], slab)
    pltpu.sync_copy(lab_hbm.at[pl.ds(SUPPORT_NUM + wid * ROWS_W, ROWS_W)],
                    tlab.at[pl.ds(0, ROWS_W)])

    zeros16 = jnp.zeros((16,), jnp.float32)
    for z in range(HIST_W // 16):
        hist[pl.ds(z * 16, 16)] = zeros16

    lane = lax.broadcasted_iota(jnp.int32, (16,), 0)
    slv = [slab[pl.ds(v * 16, 16)] for v in range(VECS_ROW)]

    def row_body(i, c):
        tl = tlab[pl.ds(i, 16)][0]
        rowoff = i * SUPPORT_NUM
        for v in range(VECS_ROW):
            s = sbuf[pl.ds(rowoff + v * 16, 16)]
            u = s * INV_STEP
            bi = u.astype(jnp.int32)
            bf = bi.astype(jnp.float32)
            frac = u - bf
            rem = (bf + 1.0) - u
            binv = jnp.minimum(jnp.maximum(bi + 64, 0), NUM_STEPS - 1)
            pos = slv[v] == tl
            base = jnp.where(pos, 0, NUM_STEPS * 16) + binv * 16 + lane
            plsc.addupdate_scatter(hist, [base], frac)
            plsc.addupdate_scatter(hist, [base + 16], rem,
                                   mask=binv < NUM_STEPS - 1)
        return c

    lax.fori_loop(0, ROWS_W, row_body, 0)
    pltpu.sync_copy(hist, out_hbm.at[wid])


@functools.cache
def _sc_hist():
    return pl.kernel(
        _sc_hist_body,
        out_type=jax.ShapeDtypeStruct((NW, HIST_W), jnp.float32),
        mesh=plsc.VectorSubcoreMesh(
            core_axis_name="c", subcore_axis_name="s",
            num_cores=NC, num_subcores=NS),
        scratch_types=[
            pltpu.VMEM((VALS_W,), jnp.float32),
            pltpu.VMEM((SUPPORT_NUM,), jnp.int32),
            pltpu.VMEM((ROWS_W + 16,), jnp.int32),
            pltpu.VMEM((HIST_W,), jnp.float32),
        ],
        compiler_params=pltpu.CompilerParams(needs_layout_passes=False),
    )


def _tc_combine_body(logits_ref, tl_ref, sl_ref, part_ref, hloss_ref, dloss_ref):
    x = logits_ref[...]
    m = jnp.max(x, axis=1, keepdims=True)
    e = jnp.exp(x - m)
    lse = jnp.log(jnp.sum(e, axis=1, keepdims=True)) + m
    cls = lax.broadcasted_iota(jnp.int32, (TARGET_NUM, CLASS_NUM), 1)
    sel = cls == tl_ref[...]
    picked = jnp.sum(jnp.where(sel, x - lse, 0.0))
    dloss_ref[...] = jnp.full((1, 1), -picked / TARGET_NUM, jnp.float32)

    tcnt = jnp.sum(sel.astype(jnp.float32), axis=0, keepdims=True)
    scls = lax.broadcasted_iota(jnp.int32, (CLASS_NUM, SUPPORT_NUM), 0)
    scnt = jnp.sum((scls == sl_ref[...]).astype(jnp.float32), axis=1,
                   keepdims=True)
    pos_num = jnp.dot(tcnt, scnt, preferred_element_type=jnp.float32)[0, 0]
    neg_num = float(TARGET_NUM * SUPPORT_NUM) - pos_num

    h = jnp.sum(jnp.sum(part_ref[...], axis=0), axis=1, keepdims=True)
    hpos = h[:NUM_STEPS] / pos_num
    hneg = h[NUM_STEPS:] / neg_num
    r = lax.broadcasted_iota(jnp.int32, (NUM_STEPS, NUM_STEPS), 0)
    c = lax.broadcasted_iota(jnp.int32, (NUM_STEPS, NUM_STEPS), 1)
    tri = (r >= c).astype(jnp.float32)
    cdf = jnp.dot(tri, hpos, preferred_element_type=jnp.float32)
    hloss_ref[...] = jnp.full((1, 1), jnp.sum(cdf * hneg), jnp.float32)


_tc_combine = pl.pallas_call(
    _tc_combine_body,
    out_shape=(
        jax.ShapeDtypeStruct((1, 1), jnp.float32),
        jax.ShapeDtypeStruct((1, 1), jnp.float32),
    ),
)


def kernel(direct_cls_logits, inner_product, cls_labels):
    labels = cls_labels.astype(jnp.int32)
    part = _sc_hist()(inner_product.reshape(-1), labels)
    part = part.reshape(NW, 2 * NUM_STEPS, 16)
    tl = labels[SUPPORT_NUM:].reshape(TARGET_NUM, 1)
    sl = labels[:SUPPORT_NUM].reshape(1, SUPPORT_NUM)
    hloss, dloss = _tc_combine(direct_cls_logits, tl, sl, part)
    return hloss[0, 0], dloss[0, 0]
